# baseline (speedup 1.0000x reference)
import functools

import jax
import jax.numpy as jnp
from jax import lax
from jax.experimental import pallas as pl
from jax.experimental.pallas import tpu as pltpu
from jax.experimental.pallas import tpu_sc as plsc

N = 8192
NW = 32
CPW = N // NW
K = 32
IDX_LEN = N + K


def _bf16_round(v):
    t = plsc.bitcast(v, jnp.int32)
    c = jnp.bitwise_and(lax.shift_right_logical(t, 16), 1)
    r = jnp.bitwise_and(t + c + 0x7FFF, jnp.int32(-65536))
    return plsc.bitcast(r, jnp.float32)


def _gather_sum_body(f_hbm, w4_hbm, out_hbm, f_v, idx_v, acc, rowbuf, sem0, sem1):
    nc = 2
    wid = lax.axis_index("s") * nc + lax.axis_index("c")

    pltpu.sync_copy(f_hbm, f_v)

    def _zi(i, _):
        idx_v[pl.ds(pl.multiple_of(i * 16, 16), 16)] = jnp.zeros((16,), jnp.int32)
        return 0
    lax.fori_loop(0, IDX_LEN // 16, _zi, 0)

    for s in range(CPW // 16):
        acc[pl.ds(s * 16, 16)] = jnp.zeros((16,), jnp.float32)

    def _cg(g, cnt):
        v = f_v[pl.ds(pl.multiple_of(g * 16, 16), 16)]
        m = v > 0.5
        inc = m.astype(jnp.int32)
        p = plsc.cumsum(inc) - inc + cnt
        j = lax.iota(jnp.int32, 16) + g * 16
        plsc.store_scatter(idx_v, [p], j * NW + wid, mask=m)
        return cnt + jnp.sum(inc)
    cnt = lax.fori_loop(0, N // 16, _cg, jnp.int32(0))

    nfull = cnt // K
    nch = (cnt + (K - 1)) // K

    def _issue(c, b, sem):
        off = pl.multiple_of(c * K, K)
        pltpu.make_async_copy(
            w4_hbm.at[idx_v.at[pl.ds(off, K)]], rowbuf.at[b], sem).start()

    def _wait(b, sem):
        pltpu.make_async_copy(
            w4_hbm.at[idx_v.at[pl.ds(0, K)]], rowbuf.at[b], sem).wait()

    def _accumulate(c, b):
        @pl.when(c < nfull)
        def _():
            def _acc_body(s, _):
                off = pl.multiple_of(s * 16, 16)
                a = acc[pl.ds(off, 16)]
                for r in range(K):
                    a = a + _bf16_round(rowbuf[b, r, pl.ds(off, 16)])
                acc[pl.ds(off, 16)] = a
                return 0
            lax.fori_loop(0, CPW // 16, _acc_body, 0)

        @pl.when(c >= nfull)
        def _():
            zero = jnp.zeros((16,), jnp.float32)

            def _acc_body(s, _):
                off = pl.multiple_of(s * 16, 16)
                a = acc[pl.ds(off, 16)]
                for r in range(K):
                    valid = jnp.full((16,), c * K + r < cnt)
                    a = a + jnp.where(
                        valid, _bf16_round(rowbuf[b, r, pl.ds(off, 16)]), zero)
                acc[pl.ds(off, 16)] = a
                return 0
            lax.fori_loop(0, CPW // 16, _acc_body, 0)

    @pl.when(nch > 0)
    def _():
        _issue(0, 0, sem0)

    @pl.when(nch > 1)
    def _():
        _issue(1, 1, sem1)

    def _body2(c2, _):
        c0 = 2 * c2
        c1 = c0 + 1

        @pl.when(c0 < nch)
        def _():
            _wait(0, sem0)
            _accumulate(c0, 0)

        @pl.when(c0 + 2 < nch)
        def _():
            _issue(c0 + 2, 0, sem0)

        @pl.when(c1 < nch)
        def _():
            _wait(1, sem1)
            _accumulate(c1, 1)

        @pl.when(c1 + 2 < nch)
        def _():
            _issue(c1 + 2, 1, sem1)

        return 0

    lax.fori_loop(0, (nch + 1) // 2, _body2, 0)

    pltpu.sync_copy(acc, out_hbm.at[pl.ds(pl.multiple_of(wid * CPW, CPW), CPW)])


def _sc_node_x(is_firing, W4):
    mesh = plsc.VectorSubcoreMesh(core_axis_name="c", subcore_axis_name="s")
    k = functools.partial(
        pl.kernel,
        mesh=mesh,
        out_type=jax.ShapeDtypeStruct((N,), jnp.float32),
        scratch_types=[
            pltpu.VMEM((N,), jnp.float32),
            pltpu.VMEM((IDX_LEN,), jnp.int32),
            pltpu.VMEM((CPW,), jnp.float32),
            pltpu.VMEM((2, K, CPW), jnp.float32),
            pltpu.SemaphoreType.DMA,
            pltpu.SemaphoreType.DMA,
        ],
        compiler_params=pltpu.CompilerParams(needs_layout_passes=False),
    )(_gather_sum_body)
    return k(is_firing, W4)


def _threshold_body(x_ref, nx_ref, nf_ref):
    v = x_ref[...]
    theta = jnp.mean(v)
    nx_ref[...] = v
    nf_ref[...] = (v > theta).astype(jnp.float32)


def kernel(x, is_firing, W):
    node_x = _sc_node_x(is_firing, W.reshape(N * NW, CPW))
    nx, nf = pl.pallas_call(
        _threshold_body,
        out_shape=(
            jax.ShapeDtypeStruct((8, N // 8), jnp.float32),
            jax.ShapeDtypeStruct((8, N // 8), jnp.float32),
        ),
    )(node_x.reshape(8, N // 8))
    return nx.reshape(N), nf.reshape(N)
